# baseline (speedup 1.0000x reference)
import functools

import jax
import jax.numpy as jnp
from jax import lax
from jax.experimental import pallas as pl
from jax.experimental.pallas import tpu as pltpu
from jax.experimental.pallas import tpu_sc as plsc

B = 16384
F = 26
V = 100000
D = 16
DENSE_IN = 13
H = 400

N = B * F
NW = 32
PER_W = N // NW
GW = 128
CHUNK = 3328
NCHUNK = PER_W // CHUNK
NG = CHUNK // GW


def _sc_gather(fm_tab, first_tab, idx):
    mesh = plsc.VectorSubcoreMesh(core_axis_name="c", subcore_axis_name="s")

    @functools.partial(
        pl.kernel,
        out_type=[
            jax.ShapeDtypeStruct((N, D), jnp.float32),
            jax.ShapeDtypeStruct((N, 1), jnp.float32),
        ],
        mesh=mesh,
        scratch_types=[
            pltpu.VMEM((CHUNK,), jnp.int32),
            pltpu.VMEM((CHUNK, D), jnp.float32),
            pltpu.VMEM((CHUNK, 1), jnp.float32),
            pltpu.SemaphoreType.DMA,
            pltpu.SemaphoreType.DMA,
        ],
        compiler_params=pltpu.CompilerParams(use_tc_tiling_on_sc=False),
    )
    def k(fm_hbm, first_hbm, idx_hbm, emb_out, first_out,
          idx_v, emb_v, first_v, sem_e, sem_f):
        wid = lax.axis_index("s") * 2 + lax.axis_index("c")
        base = wid * PER_W

        def chunk_body(c, _):
            off = base + c * CHUNK
            pltpu.sync_copy(idx_hbm.at[pl.ds(off, CHUNK)], idx_v)
            pltpu.async_copy(fm_hbm.at[idx_v], emb_v, sem_e)
            pltpu.make_async_copy(fm_hbm.at[idx_v], emb_v, sem_e).wait()
            pltpu.sync_copy(emb_v, emb_out.at[pl.ds(off, CHUNK)])
            return _

        lax.fori_loop(0, NCHUNK, chunk_body, 0)

    return k(fm_tab, first_tab, idx)


BT = 512


def _tc_body(dense_ref, emb_ref, first_ref, wd_ref, w1a_ref, w1b_ref, b1_ref,
             w2_ref, b2_ref, wout_ref, smat_ref, bias_ref, out_ref):
    dense = dense_ref[...]
    emb = emb_ref[...]
    first = first_ref[...]

    linear = jnp.dot(dense, wd_ref[...],
                     preferred_element_type=jnp.float32)[:, 0]
    linear = linear + jnp.sum(first, axis=1) + bias_ref[0]

    s = jnp.dot(emb, smat_ref[...], preferred_element_type=jnp.float32)
    fm = 0.5 * (jnp.sum(s * s, axis=1) - jnp.sum(emb * emb, axis=1))

    h = jnp.dot(dense, w1a_ref[...], preferred_element_type=jnp.float32)
    h = h + jnp.dot(emb, w1b_ref[...], preferred_element_type=jnp.float32)
    h = jnp.maximum(h + b1_ref[...], 0.0)
    h = jnp.dot(h, w2_ref[...], preferred_element_type=jnp.float32)
    h = jnp.maximum(h + b2_ref[...], 0.0)
    deep = jnp.dot(h, wout_ref[...], preferred_element_type=jnp.float32)[:, 0]

    out_ref[...] = linear + fm + deep


def _tc_mlp(dense_x, emb, first, W_dense, W1a, W1b, b1, W2, b2, W_out,
            smat, bias_sum):
    grid = (B // BT,)
    full = lambda i: (0, 0)
    return pl.pallas_call(
        _tc_body,
        grid=grid,
        in_specs=[
            pl.BlockSpec((BT, DENSE_IN), lambda i: (i, 0)),
            pl.BlockSpec((BT, F * D), lambda i: (i, 0)),
            pl.BlockSpec((BT, F), lambda i: (i, 0)),
            pl.BlockSpec((DENSE_IN, 1), full),
            pl.BlockSpec((DENSE_IN, H), full),
            pl.BlockSpec((F * D, H), full),
            pl.BlockSpec((1, H), full),
            pl.BlockSpec((H, H), full),
            pl.BlockSpec((1, H), full),
            pl.BlockSpec((H, 1), full),
            pl.BlockSpec((F * D, D), full),
            pl.BlockSpec(memory_space=pltpu.SMEM),
        ],
        out_specs=pl.BlockSpec((BT,), lambda i: (i,)),
        out_shape=jax.ShapeDtypeStruct((B,), jnp.float32),
    )(dense_x, emb, first, W_dense, W1a, W1b, b1, W2, b2, W_out, smat,
      bias_sum)


def kernel(cat_x, dense_x, W_first, W_fm, W_dense, b_dense, W1, b1, W2, b2,
           W_out, b_out, bias):
    VP = 100032
    idx = (cat_x.astype(jnp.int32)
           + (jnp.arange(F, dtype=jnp.int32) * VP)[None, :]).reshape(N)
    fm_tab = jnp.pad(W_fm, ((0, 0), (0, VP - V), (0, 0))).reshape(F * VP, D)
    first_tab = jnp.pad(W_first, ((0, 0), (0, VP - V), (0, 0))).reshape(F * VP, 1)

    emb_flat, first_flat = _sc_gather(fm_tab, first_tab, idx)
    first_flat = first_tab[idx]
    emb = emb_flat.reshape(B, F * D)
    first = first_flat.reshape(B, F)

    smat = (jnp.arange(F * D, dtype=jnp.int32)[:, None] % D
            == jnp.arange(D, dtype=jnp.int32)[None, :]).astype(jnp.float32)
    bias_sum = (bias + b_dense + b_out).reshape(1)
    W1a = W1[:DENSE_IN]
    W1b = W1[DENSE_IN:]

    if True:
        linear = (bias + dense_x @ W_dense + b_dense)[:, 0] + first.sum(axis=1)
        s = emb @ smat
        fm = 0.5 * ((s * s).sum(axis=1) - (emb * emb).sum(axis=1))
        h = jax.nn.relu(dense_x @ W1a + emb @ W1b + b1)
        h = jax.nn.relu(h @ W2 + b2)
        deep = (h @ W_out + b_out)[:, 0]
        return linear + fm + deep
    return _tc_mlp(dense_x, emb, first, W_dense, W1a, W1b,
                   b1.reshape(1, H), W2, b2.reshape(1, H), W_out,
                   smat, bias_sum)


# speedup vs baseline: 5.2154x; 5.2154x over previous
import functools

import jax
import jax.numpy as jnp
from jax import lax
from jax.experimental import pallas as pl
from jax.experimental.pallas import tpu as pltpu
from jax.experimental.pallas import tpu_sc as plsc

B = 16384
F = 26
V = 100000
D = 16
DENSE_IN = 13
H = 400

NW = 32
NCH = 16
CB = B // NCH
NTASK = F * NCH
PER_W = NTASK // NW


def _sc_gather(fm_tab, first_tab, idxT):
    mesh = plsc.VectorSubcoreMesh(core_axis_name="c", subcore_axis_name="s")

    @functools.partial(
        pl.kernel,
        out_type=[
            jax.ShapeDtypeStruct((B, F * D), jnp.float32),
            jax.ShapeDtypeStruct((F, B), jnp.float32),
        ],
        mesh=mesh,
        scratch_types=[
            pltpu.VMEM((CB,), jnp.int32),
            pltpu.VMEM((CB, D), jnp.float32),
            pltpu.VMEM((CB,), jnp.float32),
            pltpu.SemaphoreType.DMA,
            pltpu.SemaphoreType.DMA,
        ],
        compiler_params=pltpu.CompilerParams(use_tc_tiling_on_sc=False),
    )
    def k(fm_hbm, first_hbm, idx_hbm, emb_out, first_out,
          idx_v, emb_v, first_v, sem_e, sem_f):
        wid = lax.axis_index("s") * 2 + lax.axis_index("c")

        def task_body(j, _):
            t = wid * PER_W + j
            f = t // NCH
            b0 = (t % NCH) * CB
            pltpu.sync_copy(idx_hbm.at[f].at[pl.ds(b0, CB)], idx_v)
            ge = pltpu.async_copy(fm_hbm.at[f].at[idx_v], emb_v, sem_e)
            gf = pltpu.async_copy(first_hbm.at[f].at[idx_v], first_v, sem_f)
            ge.wait()
            gf.wait()
            pltpu.sync_copy(emb_v,
                            emb_out.at[pl.ds(b0, CB), pl.ds(f * D, D)])
            pltpu.sync_copy(first_v, first_out.at[f].at[pl.ds(b0, CB)])
            return _

        lax.fori_loop(0, PER_W, task_body, 0)

    return k(fm_tab, first_tab, idxT)


BT = 512


def _tc_body(dense_ref, emb_ref, first_ref, wd_ref, w1a_ref, w1b_ref, b1_ref,
             w2_ref, b2_ref, wout_ref, smat_ref, bias_ref, out_ref):
    dense = dense_ref[...]
    emb = emb_ref[...]
    first = first_ref[...]

    linear = jnp.dot(dense, wd_ref[...],
                     preferred_element_type=jnp.float32)[:, 0]
    linear = linear + jnp.sum(first, axis=0) + bias_ref[0]

    s = jnp.dot(emb, smat_ref[...], preferred_element_type=jnp.float32)
    fm = 0.5 * (jnp.sum(s * s, axis=1) - jnp.sum(emb * emb, axis=1))

    h = jnp.dot(dense, w1a_ref[...], preferred_element_type=jnp.float32)
    h = h + jnp.dot(emb, w1b_ref[...], preferred_element_type=jnp.float32)
    h = jnp.maximum(h + b1_ref[...], 0.0)
    h = jnp.dot(h, w2_ref[...], preferred_element_type=jnp.float32)
    h = jnp.maximum(h + b2_ref[...], 0.0)
    deep = jnp.dot(h, wout_ref[...], preferred_element_type=jnp.float32)[:, 0]

    out_ref[...] = linear + fm + deep


def _tc_mlp(dense_x, emb, firstT, W_dense, W1a, W1b, b1, W2, b2, W_out,
            smat, bias_sum):
    grid = (B // BT,)
    full = lambda i: (0, 0)
    return pl.pallas_call(
        _tc_body,
        grid=grid,
        in_specs=[
            pl.BlockSpec((BT, DENSE_IN), lambda i: (i, 0)),
            pl.BlockSpec((BT, F * D), lambda i: (i, 0)),
            pl.BlockSpec((F, BT), lambda i: (0, i)),
            pl.BlockSpec((DENSE_IN, 1), full),
            pl.BlockSpec((DENSE_IN, H), full),
            pl.BlockSpec((F * D, H), full),
            pl.BlockSpec((1, H), full),
            pl.BlockSpec((H, H), full),
            pl.BlockSpec((1, H), full),
            pl.BlockSpec((H, 1), full),
            pl.BlockSpec((F * D, D), full),
            pl.BlockSpec(memory_space=pltpu.SMEM),
        ],
        out_specs=pl.BlockSpec((BT,), lambda i: (i,)),
        out_shape=jax.ShapeDtypeStruct((B,), jnp.float32),
    )(dense_x, emb, firstT, W_dense, W1a, W1b, b1, W2, b2, W_out, smat,
      bias_sum)


def kernel(cat_x, dense_x, W_first, W_fm, W_dense, b_dense, W1, b1, W2, b2,
           W_out, b_out, bias):
    idxT = cat_x.astype(jnp.int32).T
    first2 = W_first.reshape(F, V)

    emb, firstT = _sc_gather(W_fm, first2, idxT)

    smat = (jnp.arange(F * D, dtype=jnp.int32)[:, None] % D
            == jnp.arange(D, dtype=jnp.int32)[None, :]).astype(jnp.float32)
    bias_sum = (bias + b_dense + b_out).reshape(1)
    W1a = W1[:DENSE_IN]
    W1b = W1[DENSE_IN:]

    return _tc_mlp(dense_x, emb, firstT, W_dense, W1a, W1b,
                   b1.reshape(1, H), W2, b2.reshape(1, H), W_out,
                   smat, bias_sum)
